# baseline (speedup 1.0000x reference)
import jax
import jax.numpy as jnp
from jax import lax
from jax.experimental import pallas as pl
from jax.experimental.pallas import tpu as pltpu
from jax.experimental.pallas import tpu_sc as plsc

_N = 16384
_FEAT = 64
_CLS = 1000
_NC = 2
_NS = 16
_NW = _NC * _NS
_RPW = _N // _NW
_CHK = 128
_NCHK = _RPW // _CHK
_BLK = 16
_HL = _N // _NS
_HB = 1024
_NSUB = 4
_L = 16


def _rsqrt(s):
    i = lax.bitcast_convert_type(s, jnp.int32)
    y = lax.bitcast_convert_type(jnp.int32(0x5F3759DF) - (i >> 1), jnp.float32)
    for _ in range(4):
        y = y * (1.5 - 0.5 * s * y * y)
    return y


def _body(x_hbm, lab_hbm, cen_hbm, out_hbm,
          xa_v, xb_v, cen_v, hlab_v, sub_v, hist_v, histc_v,
          stripe_v, tmp_v, tree_v, acc1_v, accall_v,
          sp_slab, sp_hist, sp_acc,
          sem_x, sem_c, sem_s):
    c = lax.axis_index("c")
    s = lax.axis_index("s")
    wid = s * _NC + c
    base = wid * _RPW
    xbufs = [xa_v, xb_v]

    cps = [pltpu.async_copy(x_hbm.at[pl.ds(base, _CHK)], xa_v, sem_x), None]
    cp_c = pltpu.async_copy(cen_hbm, cen_v, sem_c)
    pltpu.sync_copy(lab_hbm.at[pl.ds(s * _HL, _HL)], hlab_v)

    zeros16 = jnp.zeros((_L,), jnp.float32)
    iota16 = lax.iota(jnp.int32, _L)
    one0 = jnp.where(iota16 == 0, 1.0, 0.0).astype(jnp.float32)

    def zero_body(i, carry):
        sub_v[pl.ds(i * _L, _L)] = zeros16
        return carry

    lax.fori_loop(0, _NSUB * _HB // _L, zero_body, 0)

    def rmw_body(g, carry):
        for i in range(_NSUB):
            lv = hlab_v[pl.ds(g * (_NSUB * _L) + i * _L, _L)]
            for k in range(_L):
                off = i * _HB + lv[k]
                sub_v[pl.ds(off, _L)] = sub_v[pl.ds(off, _L)] + one0
        return carry

    lax.fori_loop(0, _HL // (_NSUB * _L), rmw_body, 0)

    def merge_body(v, carry):
        a = sub_v[pl.ds(v * _L, _L)]
        for i in range(1, _NSUB):
            a = a + sub_v[pl.ds(i * _HB + v * _L, _L)]
        hist_v[pl.ds(v * _L, _L)] = a
        return carry

    lax.fori_loop(0, _HB // _L, merge_body, 0)

    pltpu.sync_copy(hist_v, sp_slab.at[s])
    plsc.subcore_barrier()
    cps_s = [pltpu.async_copy(sp_slab.at[r, pl.ds(s * 64, 64)],
                              stripe_v.at[r], sem_s)
             for r in range(_NS)]
    for cp in cps_s:
        cp.wait()
    for j in range(4):
        a = stripe_v[0, pl.ds(j * _L, _L)]
        for r in range(1, _NS):
            a = a + stripe_v[r, pl.ds(j * _L, _L)]
        tmp_v[pl.ds(j * _L, _L)] = a
    pltpu.sync_copy(tmp_v, sp_hist.at[pl.ds(s * 64, 64)])
    plsc.subcore_barrier()
    pltpu.sync_copy(sp_hist, histc_v)

    cp_c.wait()

    total = zeros16
    for q in range(_NCHK):
        cps[q % 2].wait()
        if q + 1 < _NCHK:
            cps[(q + 1) % 2] = pltpu.async_copy(
                x_hbm.at[pl.ds(base + (q + 1) * _CHK, _CHK)],
                xbufs[(q + 1) % 2], sem_x)
        xb = xbufs[q % 2]

        def blk(bb, tot, q=q, xb=xb):
            row0 = q * _CHK
            lv = hlab_v[pl.ds(c * _RPW + row0 + bb * _BLK, _L)]
            ssum = zeros16
            cnt = zeros16
            for r in range(_BLK):
                xrow = bb * _BLK + r
                l = lv[r]
                lrow = l >> 1
                loff = (l & 1) * 64
                a = zeros16
                for j in range(4):
                    xv = xb[xrow, pl.ds(j * _L, _L)]
                    cv = cen_v[lrow, pl.ds(loff + j * _L, _L)]
                    d = xv - cv
                    a = a + d * d
                tb = r * 32
                tree_v[pl.ds(tb, _L)] = a
                v = a + tree_v[pl.ds(tb + 8, _L)]
                tree_v[pl.ds(tb, _L)] = v
                v = v + tree_v[pl.ds(tb + 4, _L)]
                tree_v[pl.ds(tb, _L)] = v
                v = v + tree_v[pl.ds(tb + 2, _L)]
                tree_v[pl.ds(tb, _L)] = v
                v = v + tree_v[pl.ds(tb + 1, _L)]
                ssum = jnp.where(iota16 == r, v[0], ssum)
                cnt = jnp.where(iota16 == r, histc_v[pl.ds(l, _L)][0], cnt)
            dist = ssum * _rsqrt(ssum)
            return tot + dist / cnt

        total = lax.fori_loop(0, _CHK // _BLK, blk, total)

    acc1_v[pl.ds(0, _L)] = total
    pltpu.sync_copy(acc1_v, sp_acc.at[pl.ds(s * _L, _L)])
    plsc.subcore_barrier()

    @pl.when(s == 0)
    def _():
        pltpu.sync_copy(sp_acc, accall_v)
        t = zeros16
        for r in range(_NS):
            t = t + accall_v[pl.ds(r * _L, _L)]
        acc1_v[pl.ds(0, _L)] = t
        pltpu.sync_copy(acc1_v, out_hbm.at[c])


@jax.jit
def _sc_loss(x, labels, center):
    mesh = plsc.VectorSubcoreMesh(core_axis_name="c", subcore_axis_name="s")
    fn = pl.kernel(
        _body,
        out_type=jax.ShapeDtypeStruct((_NC, _L), jnp.float32),
        mesh=mesh,
        compiler_params=pltpu.CompilerParams(use_tc_tiling_on_sc=True),
        scratch_types=[
            pltpu.VMEM((_CHK, _FEAT), jnp.float32),
            pltpu.VMEM((_CHK, _FEAT), jnp.float32),
            pltpu.VMEM((_CLS // 2, 128), jnp.float32),
            pltpu.VMEM((_HL,), jnp.int32),
            pltpu.VMEM((_NSUB * _HB,), jnp.float32),
            pltpu.VMEM((_HB,), jnp.float32),
            pltpu.VMEM((_HB,), jnp.float32),
            pltpu.VMEM((_NS, 64), jnp.float32),
            pltpu.VMEM((64,), jnp.float32),
            pltpu.VMEM((_BLK * 32,), jnp.float32),
            pltpu.VMEM((_L,), jnp.float32),
            pltpu.VMEM((_NS * _L,), jnp.float32),
            pltpu.VMEM_SHARED((_NS, _HB), jnp.float32),
            pltpu.VMEM_SHARED((_HB,), jnp.float32),
            pltpu.VMEM_SHARED((_NS * _L,), jnp.float32),
            pltpu.SemaphoreType.DMA,
            pltpu.SemaphoreType.DMA,
            pltpu.SemaphoreType.DMA,
        ],
    )
    return fn(x, labels, center.reshape(_CLS // 2, 128))


def kernel(x, labels, center):
    out = _sc_loss(x, labels, center)
    return jnp.sum(out)


# speedup vs baseline: 1.3855x; 1.3855x over previous
import jax
import jax.numpy as jnp
from jax import lax
from jax.experimental import pallas as pl
from jax.experimental.pallas import tpu as pltpu
from jax.experimental.pallas import tpu_sc as plsc

_N = 16384
_FEAT = 64
_CLS = 1000
_NC = 2
_NS = 16
_NW = _NC * _NS
_RPW = _N // _NW
_CHK = 128
_NCHK = _RPW // _CHK
_BLK = 16
_HL = _N // _NS
_HB = 1024
_NSUB = 4
_L = 16


def _rsqrt(s):
    i = lax.bitcast_convert_type(s, jnp.int32)
    y = lax.bitcast_convert_type(jnp.int32(0x5F3759DF) - (i >> 1), jnp.float32)
    for _ in range(4):
        y = y * (1.5 - 0.5 * s * y * y)
    return y


def _body(x_hbm, lab_hbm, cen_hbm, out_hbm,
          xa_v, xb_v, cen_v, hlab_v, sub_v, hist_v, histc_v,
          stripe_v, tmp_v, tree_v, acc1_v, accall_v,
          sp_slab, sp_hist, sp_acc,
          sem_x, sem_c, sem_s):
    c = lax.axis_index("c")
    s = lax.axis_index("s")
    wid = s * _NC + c
    base = wid * _RPW
    xbufs = [xa_v, xb_v]

    cps = [pltpu.async_copy(x_hbm.at[pl.ds(base, _CHK)], xa_v, sem_x), None]
    cp_c = pltpu.async_copy(cen_hbm, cen_v, sem_c)
    pltpu.sync_copy(lab_hbm.at[pl.ds(s * _HL, _HL)], hlab_v)

    zeros16 = jnp.zeros((_L,), jnp.float32)
    iota16 = lax.iota(jnp.int32, _L)
    one0 = jnp.where(iota16 == 0, 1.0, 0.0).astype(jnp.float32)

    def zero_body(i, carry):
        sub_v[pl.ds(i * _L, _L)] = zeros16
        return carry

    lax.fori_loop(0, _NSUB * _HB // _L, zero_body, 0)

    def rmw_body(g, carry):
        for i in range(_NSUB):
            lv = hlab_v[pl.ds(g * (_NSUB * _L) + i * _L, _L)]
            for k in range(_L):
                off = i * _HB + lv[k]
                sub_v[pl.ds(off, _L)] = sub_v[pl.ds(off, _L)] + one0
        return carry

    lax.fori_loop(0, _HL // (_NSUB * _L), rmw_body, 0)

    def merge_body(v, carry):
        a = sub_v[pl.ds(v * _L, _L)]
        for i in range(1, _NSUB):
            a = a + sub_v[pl.ds(i * _HB + v * _L, _L)]
        hist_v[pl.ds(v * _L, _L)] = a
        return carry

    lax.fori_loop(0, _HB // _L, merge_body, 0)

    pltpu.sync_copy(hist_v, sp_slab.at[s])
    plsc.subcore_barrier()
    cps_s = [pltpu.async_copy(sp_slab.at[r, pl.ds(s * 64, 64)],
                              stripe_v.at[r], sem_s)
             for r in range(_NS)]
    for cp in cps_s:
        cp.wait()
    for j in range(4):
        a = stripe_v[0, pl.ds(j * _L, _L)]
        for r in range(1, _NS):
            a = a + stripe_v[r, pl.ds(j * _L, _L)]
        tmp_v[pl.ds(j * _L, _L)] = a
    pltpu.sync_copy(tmp_v, sp_hist.at[pl.ds(s * 64, 64)])
    plsc.subcore_barrier()
    pltpu.sync_copy(sp_hist, histc_v)

    cp_c.wait()

    total = zeros16
    for q in range(_NCHK):
        cps[q % 2].wait()
        if q + 1 < _NCHK:
            cps[(q + 1) % 2] = pltpu.async_copy(
                x_hbm.at[pl.ds(base + (q + 1) * _CHK, _CHK)],
                xbufs[(q + 1) % 2], sem_x)
        xb = xbufs[q % 2]

        def blk(bb, tot, q=q, xb=xb):
            row0 = q * _CHK
            lv = hlab_v[pl.ds(c * _RPW + row0 + bb * _BLK, _L)]
            ssum = zeros16
            cnt = zeros16
            for r in range(_BLK):
                xrow = bb * _BLK + r
                l = lv[r]
                lrow = l >> 1
                loff = (l & 1) * 64
                a = zeros16
                for j in range(4):
                    xv = xb[xrow, pl.ds(j * _L, _L)]
                    cv = cen_v[lrow, pl.ds(loff + j * _L, _L)]
                    d = xv - cv
                    a = a + d * d
                ssum = jnp.where(iota16 == r, jnp.sum(a), ssum)
            cnt = plsc.load_gather(histc_v, [lv])
            dist = ssum * _rsqrt(ssum)
            return tot + dist / cnt

        total = lax.fori_loop(0, _CHK // _BLK, blk, total)

    acc1_v[pl.ds(0, _L)] = total
    pltpu.sync_copy(acc1_v, sp_acc.at[pl.ds(s * _L, _L)])
    plsc.subcore_barrier()

    @pl.when(s == 0)
    def _():
        pltpu.sync_copy(sp_acc, accall_v)
        t = zeros16
        for r in range(_NS):
            t = t + accall_v[pl.ds(r * _L, _L)]
        acc1_v[pl.ds(0, _L)] = t
        pltpu.sync_copy(acc1_v, out_hbm.at[c])


@jax.jit
def _sc_loss(x, labels, center):
    mesh = plsc.VectorSubcoreMesh(core_axis_name="c", subcore_axis_name="s")
    fn = pl.kernel(
        _body,
        out_type=jax.ShapeDtypeStruct((_NC, _L), jnp.float32),
        mesh=mesh,
        compiler_params=pltpu.CompilerParams(use_tc_tiling_on_sc=True,
                                             needs_layout_passes=False),
        scratch_types=[
            pltpu.VMEM((_CHK, _FEAT), jnp.float32),
            pltpu.VMEM((_CHK, _FEAT), jnp.float32),
            pltpu.VMEM((_CLS // 2, 128), jnp.float32),
            pltpu.VMEM((_HL,), jnp.int32),
            pltpu.VMEM((_NSUB * _HB,), jnp.float32),
            pltpu.VMEM((_HB,), jnp.float32),
            pltpu.VMEM((_HB,), jnp.float32),
            pltpu.VMEM((_NS, 64), jnp.float32),
            pltpu.VMEM((64,), jnp.float32),
            pltpu.VMEM((_BLK * 32,), jnp.float32),
            pltpu.VMEM((_L,), jnp.float32),
            pltpu.VMEM((_NS * _L,), jnp.float32),
            pltpu.VMEM_SHARED((_NS, _HB), jnp.float32),
            pltpu.VMEM_SHARED((_HB,), jnp.float32),
            pltpu.VMEM_SHARED((_NS * _L,), jnp.float32),
            pltpu.SemaphoreType.DMA,
            pltpu.SemaphoreType.DMA,
            pltpu.SemaphoreType.DMA,
        ],
    )
    return fn(x, labels, center.reshape(_CLS // 2, 128))


def kernel(x, labels, center):
    out = _sc_loss(x, labels, center)
    return jnp.sum(out)
